# baseline (speedup 1.0000x reference)
import jax
import jax.numpy as jnp
from jax import lax
from jax.experimental import pallas as pl
from jax.experimental.pallas import tpu as pltpu

_SLAB = 8
_VMAX = 20480
_TN = 128
_W = _TN + 8
_VACC = 20736


def _ceil_to(x, m):
    return (x + m - 1) // m * m


def _sum_kernel(base_ref, slab_ref, seg_ref, out_ref):
    i = pl.program_id(0)
    j = pl.program_id(1)
    half = pl.num_programs(1)
    tn = slab_ref.shape[1]
    w = tn + 8

    @pl.when(j == 0)
    def _():
        out_ref[...] = jnp.zeros_like(out_ref)

    base = base_ref[i * half + j]
    seg = seg_ref[...]
    win = lax.broadcasted_iota(jnp.int32, (w, tn), 0) + base
    mask = (win == seg).astype(jnp.float32)
    psum = jnp.dot(mask, slab_ref[...].T,
                   preferred_element_type=jnp.float32)
    b8 = pl.multiple_of(base, 8)
    out_ref[0, pl.ds(b8, w), :] += psum


def _pfn_kernel(base_ref, slab_ref, seg_ref, first_ref, sums_ref,
                wpt_ref, wmn_ref, out_ref, mean_ref):
    i = pl.program_id(0)
    j = pl.program_id(1)
    half = pl.num_programs(1)
    tn = slab_ref.shape[1]
    w = tn + 8

    @pl.when(j == 0)
    def _():
        s = sums_ref[0] + sums_ref[1]
        cnt = jnp.maximum(s[:, _SLAB - 1:_SLAB], 1.0)
        mean_ref[...] = s / cnt
        out_ref[...] = jnp.zeros_like(out_ref)

    base = base_ref[i * half + j]
    b8 = pl.multiple_of(base, 8)
    slab = slab_ref[...]
    seg = seg_ref[...]
    win = lax.broadcasted_iota(jnp.int32, (w, tn), 0) + base
    mask = (win == seg).astype(jnp.float32)

    mean_t = mean_ref[pl.ds(b8, w), :].T
    pm = jnp.dot(mean_t, mask, preferred_element_type=jnp.float32)
    pf = jnp.maximum(
        jnp.dot(wpt_ref[...], slab, preferred_element_type=jnp.float32)
        + jnp.dot(wmn_ref[...], pm, preferred_element_type=jnp.float32),
        0.0)

    k = 1
    while k < tn:
        sh = tn - k
        same = seg == pltpu.roll(seg, shift=sh, axis=1)
        pf = jnp.where(same,
                       jnp.maximum(pf, pltpu.roll(pf, shift=sh, axis=1)),
                       pf)
        k *= 2

    lane = lax.broadcasted_iota(jnp.int32, (1, tn), 1)
    sel = jnp.logical_or(first_ref[...] != 0, lane == 0)
    pf_sel = jnp.where(sel, pf, 0.0)

    loc = jnp.dot(mask, pf_sel.T,
                  preferred_element_type=jnp.float32)
    cur = out_ref[0, pl.ds(b8, w), :]
    out_ref[0, pl.ds(b8, w), :] = jnp.maximum(cur, loc)


def _dpfn(features, coors, w_t, bn_scale, bn_shift, *,
          voxel_size, point_cloud_range, v_max, v_acc, tn):
    n = features.shape[0]
    c_out = w_t.shape[1]
    w = tn + 8

    vx, vy, vz = (float(s) for s in voxel_size)
    x_off = vx / 2.0 + float(point_cloud_range[0])
    y_off = vy / 2.0 + float(point_cloud_range[1])
    z_off = vz / 2.0 + float(point_cloud_range[2])
    canvas_y = int(round((point_cloud_range[4] - point_cloud_range[1]) / vy))
    canvas_x = int(round((point_cloud_range[3] - point_cloud_range[0]) / vx))

    coors = coors.astype(jnp.int32)
    feats = features.astype(jnp.float32)

    vid = (coors[:, 0] * canvas_y * canvas_x
           + coors[:, 2] * canvas_x + coors[:, 3])
    order = jnp.argsort(vid)
    vid_s = vid[order]
    coors_s = coors[order]
    feats_s = feats[order]
    is_new = jnp.concatenate(
        [jnp.ones((1,), jnp.bool_), vid_s[1:] != vid_s[:-1]])
    seg = jnp.cumsum(is_new.astype(jnp.int32)) - 1

    idx = jnp.arange(n, dtype=jnp.int32)
    first_idx = jax.ops.segment_min(
        idx, jnp.minimum(seg, v_max), num_segments=v_max + 1)[:v_max]
    occupied = first_idx < n
    voxel_coors = jnp.where(occupied[:, None],
                            coors_s[jnp.clip(first_idx, 0, n - 1)],
                            jnp.int32(-1))

    fcx = feats_s[:, 0] - (coors_s[:, 3].astype(jnp.float32) * vx + x_off)
    fcy = feats_s[:, 1] - (coors_s[:, 2].astype(jnp.float32) * vy + y_off)
    fcz = feats_s[:, 2] - (coors_s[:, 1].astype(jnp.float32) * vz + z_off)
    slab_t = jnp.stack([feats_s[:, 0], feats_s[:, 1], feats_s[:, 2],
                        feats_s[:, 3], fcx, fcy, fcz,
                        jnp.ones((n,), jnp.float32)], axis=0)

    bn_scale = bn_scale.reshape(1, -1).astype(jnp.float32)
    bn_shift = bn_shift.reshape(-1).astype(jnp.float32)
    w_eff = w_t.astype(jnp.float32) * bn_scale
    w_pt = jnp.stack([w_eff[0] + w_eff[4], w_eff[1] + w_eff[5],
                      w_eff[2] + w_eff[6], w_eff[3],
                      w_eff[7], w_eff[8], w_eff[9], bn_shift], axis=0).T
    w_mn = jnp.concatenate(
        [-w_eff[4:7], jnp.zeros((_SLAB - 3, c_out), jnp.float32)], axis=0).T

    n_pad = _ceil_to(max(n, tn), 2 * tn)
    sentinel = jnp.int32(1 << 24)
    slab_t = jnp.pad(slab_t, ((0, 0), (0, n_pad - n)))
    seg_p = jnp.pad(seg, (0, n_pad - n), constant_values=sentinel)
    seg_row = seg_p.reshape(1, n_pad)
    first_row = jnp.pad(is_new.astype(jnp.int32),
                        (0, n_pad - n)).reshape(1, n_pad)
    bases = jnp.minimum((seg_p[::tn] // 8) * 8, v_max).astype(jnp.int32)

    tiles = n_pad // tn
    grid = (2, tiles // 2)
    cparams = pltpu.CompilerParams(
        dimension_semantics=("parallel", "arbitrary"),
        vmem_limit_bytes=64 * 1024 * 1024)

    base_spec = pl.BlockSpec(memory_space=pltpu.SMEM)
    slab_spec = pl.BlockSpec((_SLAB, tn), lambda i, j: (0, i * (tiles // 2) + j))
    seg_spec = pl.BlockSpec((1, tn), lambda i, j: (0, i * (tiles // 2) + j))
    acc8_spec = pl.BlockSpec((1, v_acc, _SLAB), lambda i, j: (i, 0, 0))
    sums_spec = pl.BlockSpec((2, v_acc, _SLAB), lambda i, j: (0, 0, 0))
    w_spec = pl.BlockSpec((c_out, _SLAB), lambda i, j: (0, 0))
    out_spec = pl.BlockSpec((1, v_acc, c_out), lambda i, j: (i, 0, 0))

    sums = pl.pallas_call(
        _sum_kernel,
        out_shape=jax.ShapeDtypeStruct((2, v_acc, _SLAB), jnp.float32),
        grid=grid,
        in_specs=[base_spec, slab_spec, seg_spec],
        out_specs=acc8_spec,
        compiler_params=cparams,
    )(bases, slab_t, seg_row)

    feats_p = pl.pallas_call(
        _pfn_kernel,
        out_shape=jax.ShapeDtypeStruct((2, v_acc, c_out), jnp.float32),
        grid=grid,
        in_specs=[base_spec, slab_spec, seg_spec, seg_spec, sums_spec,
                  w_spec, w_spec],
        out_specs=out_spec,
        scratch_shapes=[pltpu.VMEM((v_acc, _SLAB), jnp.float32)],
        compiler_params=cparams,
    )(bases, slab_t, seg_row, first_row, sums, w_pt, w_mn)

    voxel_feats = jnp.maximum(feats_p[0], feats_p[1])[:v_max]
    return voxel_feats, voxel_coors


def kernel(features, coors, w_t, bn_scale, bn_shift):
    return _dpfn(features, coors, w_t, bn_scale, bn_shift,
                 voxel_size=(0.16, 0.16, 4.0),
                 point_cloud_range=(0.0, -6.4, -3.0, 20.48, 6.4, 1.0),
                 v_max=_VMAX, v_acc=_VACC, tn=_TN)


# speedup vs baseline: 2.2957x; 2.2957x over previous
import jax
import jax.numpy as jnp
from jax import lax
from jax.experimental import pallas as pl
from jax.experimental.pallas import tpu as pltpu

_SLAB = 8
_VMAX = 20480
_TN = 128
_SUB = 16
_VACC = 20736


def _ceil_to(x, m):
    return (x + m - 1) // m * m


def _sum_kernel(base_ref, slab_ref, seg_ref, out_ref):
    i = pl.program_id(0)
    j = pl.program_id(1)
    half = pl.num_programs(1)
    tnb = slab_ref.shape[1]
    sub = tnb // _TN
    w = _TN + 8

    @pl.when(j == 0)
    def _():
        out_ref[...] = jnp.zeros_like(out_ref)

    for k in range(sub):
        sl = slab_ref[:, k * _TN:(k + 1) * _TN]
        sg = seg_ref[:, k * _TN:(k + 1) * _TN]
        base = base_ref[(i * half + j) * sub + k]
        win = lax.broadcasted_iota(jnp.int32, (w, _TN), 0) + base
        mask = (win == sg).astype(jnp.float32)
        psum = jnp.dot(mask, sl.T, preferred_element_type=jnp.float32)
        b8 = pl.multiple_of(base, 8)
        out_ref[0, pl.ds(b8, w), :] += psum


def _pfn_kernel(base_ref, slab_ref, seg_ref, first_ref, sums_ref,
                wpt_ref, wmn_ref, out_ref, mean_ref):
    i = pl.program_id(0)
    j = pl.program_id(1)
    half = pl.num_programs(1)
    tnb = slab_ref.shape[1]
    sub = tnb // _TN
    w = _TN + 8

    @pl.when(j == 0)
    def _():
        s = sums_ref[0] + sums_ref[1]
        cnt = jnp.maximum(s[:, _SLAB - 1:_SLAB], 1.0)
        mean_ref[...] = s / cnt
        out_ref[...] = jnp.zeros_like(out_ref)

    for k in range(sub):
        sl = slab_ref[:, k * _TN:(k + 1) * _TN]
        sg = seg_ref[:, k * _TN:(k + 1) * _TN]
        fr = first_ref[:, k * _TN:(k + 1) * _TN]
        base = base_ref[(i * half + j) * sub + k]
        b8 = pl.multiple_of(base, 8)
        win = lax.broadcasted_iota(jnp.int32, (w, _TN), 0) + base
        mask = (win == sg).astype(jnp.float32)

        mean_t = mean_ref[pl.ds(b8, w), :].T
        pm = jnp.dot(mean_t, mask, preferred_element_type=jnp.float32)
        pf = jnp.maximum(
            jnp.dot(wpt_ref[...], sl, preferred_element_type=jnp.float32)
            + jnp.dot(wmn_ref[...], pm, preferred_element_type=jnp.float32),
            0.0)

        kk = 1
        while kk < _TN:
            sh = _TN - kk
            same = sg == pltpu.roll(sg, shift=sh, axis=1)
            pf = jnp.where(same,
                           jnp.maximum(pf, pltpu.roll(pf, shift=sh, axis=1)),
                           pf)
            kk *= 2

        lane = lax.broadcasted_iota(jnp.int32, (1, _TN), 1)
        sel = jnp.logical_or(fr != 0, lane == 0)
        pf_sel = jnp.where(sel, pf, 0.0)

        loc = jnp.dot(mask, pf_sel.T,
                      preferred_element_type=jnp.float32)
        cur = out_ref[0, pl.ds(b8, w), :]
        out_ref[0, pl.ds(b8, w), :] = jnp.maximum(cur, loc)


def _dpfn(features, coors, w_t, bn_scale, bn_shift, *,
          voxel_size, point_cloud_range, v_max, v_acc, tn, sub):
    n = features.shape[0]
    c_out = w_t.shape[1]
    tnb = tn * sub

    vx, vy, vz = (float(s) for s in voxel_size)
    x_off = vx / 2.0 + float(point_cloud_range[0])
    y_off = vy / 2.0 + float(point_cloud_range[1])
    z_off = vz / 2.0 + float(point_cloud_range[2])
    canvas_y = int(round((point_cloud_range[4] - point_cloud_range[1]) / vy))
    canvas_x = int(round((point_cloud_range[3] - point_cloud_range[0]) / vx))

    coors = coors.astype(jnp.int32)
    feats = features.astype(jnp.float32)

    vid = (coors[:, 0] * canvas_y * canvas_x
           + coors[:, 2] * canvas_x + coors[:, 3])
    order = jnp.argsort(vid)
    vid_s = vid[order]
    coors_s = coors[order]
    feats_s = feats[order]
    is_new = jnp.concatenate(
        [jnp.ones((1,), jnp.bool_), vid_s[1:] != vid_s[:-1]])
    seg = jnp.cumsum(is_new.astype(jnp.int32)) - 1

    idx = jnp.arange(n, dtype=jnp.int32)
    first_idx = jax.ops.segment_min(
        idx, jnp.minimum(seg, v_max), num_segments=v_max + 1)[:v_max]
    occupied = first_idx < n
    voxel_coors = jnp.where(occupied[:, None],
                            coors_s[jnp.clip(first_idx, 0, n - 1)],
                            jnp.int32(-1))

    fcx = feats_s[:, 0] - (coors_s[:, 3].astype(jnp.float32) * vx + x_off)
    fcy = feats_s[:, 1] - (coors_s[:, 2].astype(jnp.float32) * vy + y_off)
    fcz = feats_s[:, 2] - (coors_s[:, 1].astype(jnp.float32) * vz + z_off)
    slab_t = jnp.stack([feats_s[:, 0], feats_s[:, 1], feats_s[:, 2],
                        feats_s[:, 3], fcx, fcy, fcz,
                        jnp.ones((n,), jnp.float32)], axis=0)

    bn_scale = bn_scale.reshape(1, -1).astype(jnp.float32)
    bn_shift = bn_shift.reshape(-1).astype(jnp.float32)
    w_eff = w_t.astype(jnp.float32) * bn_scale
    w_pt = jnp.stack([w_eff[0] + w_eff[4], w_eff[1] + w_eff[5],
                      w_eff[2] + w_eff[6], w_eff[3],
                      w_eff[7], w_eff[8], w_eff[9], bn_shift], axis=0).T
    w_mn = jnp.concatenate(
        [-w_eff[4:7], jnp.zeros((_SLAB - 3, c_out), jnp.float32)], axis=0).T

    n_pad = _ceil_to(max(n, tnb), 2 * tnb)
    sentinel = jnp.int32(1 << 24)
    slab_t = jnp.pad(slab_t, ((0, 0), (0, n_pad - n)))
    seg_p = jnp.pad(seg, (0, n_pad - n), constant_values=sentinel)
    seg_row = seg_p.reshape(1, n_pad)
    first_row = jnp.pad(is_new.astype(jnp.int32),
                        (0, n_pad - n)).reshape(1, n_pad)
    bases = jnp.minimum((seg_p[::tn] // 8) * 8, v_max).astype(jnp.int32)

    blocks = n_pad // tnb
    grid = (2, blocks // 2)
    cparams = pltpu.CompilerParams(
        dimension_semantics=("parallel", "arbitrary"),
        vmem_limit_bytes=64 * 1024 * 1024)

    base_spec = pl.BlockSpec(memory_space=pltpu.SMEM)
    slab_spec = pl.BlockSpec((_SLAB, tnb),
                             lambda i, j: (0, i * (blocks // 2) + j))
    seg_spec = pl.BlockSpec((1, tnb), lambda i, j: (0, i * (blocks // 2) + j))
    acc8_spec = pl.BlockSpec((1, v_acc, _SLAB), lambda i, j: (i, 0, 0))
    sums_spec = pl.BlockSpec((2, v_acc, _SLAB), lambda i, j: (0, 0, 0))
    w_spec = pl.BlockSpec((c_out, _SLAB), lambda i, j: (0, 0))
    out_spec = pl.BlockSpec((1, v_acc, c_out), lambda i, j: (i, 0, 0))

    sums = pl.pallas_call(
        _sum_kernel,
        out_shape=jax.ShapeDtypeStruct((2, v_acc, _SLAB), jnp.float32),
        grid=grid,
        in_specs=[base_spec, slab_spec, seg_spec],
        out_specs=acc8_spec,
        compiler_params=cparams,
    )(bases, slab_t, seg_row)

    feats_p = pl.pallas_call(
        _pfn_kernel,
        out_shape=jax.ShapeDtypeStruct((2, v_acc, c_out), jnp.float32),
        grid=grid,
        in_specs=[base_spec, slab_spec, seg_spec, seg_spec, sums_spec,
                  w_spec, w_spec],
        out_specs=out_spec,
        scratch_shapes=[pltpu.VMEM((v_acc, _SLAB), jnp.float32)],
        compiler_params=cparams,
    )(bases, slab_t, seg_row, first_row, sums, w_pt, w_mn)

    voxel_feats = jnp.maximum(feats_p[0], feats_p[1])[:v_max]
    return voxel_feats, voxel_coors


def kernel(features, coors, w_t, bn_scale, bn_shift):
    return _dpfn(features, coors, w_t, bn_scale, bn_shift,
                 voxel_size=(0.16, 0.16, 4.0),
                 point_cloud_range=(0.0, -6.4, -3.0, 20.48, 6.4, 1.0),
                 v_max=_VMAX, v_acc=_VACC, tn=_TN, sub=_SUB)


# speedup vs baseline: 2.6841x; 1.1692x over previous
import jax
import jax.numpy as jnp
from jax import lax
from jax.experimental import pallas as pl
from jax.experimental.pallas import tpu as pltpu

_SLAB = 8
_VMAX = 20480
_TN = 128
_SUB = 16
_VACC = 20736


def _ceil_to(x, m):
    return (x + m - 1) // m * m


def _sum_kernel(base_ref, slab_ref, seg_ref, out_ref, acc1):
    i = pl.program_id(0)
    j = pl.program_id(1)
    half = pl.num_programs(1)
    tnb = slab_ref.shape[1]
    sub = tnb // _TN
    w = _TN + 8

    @pl.when(j == 0)
    def _():
        out_ref[...] = jnp.zeros_like(out_ref)
        acc1[...] = jnp.zeros_like(acc1)

    for k in range(sub):
        sl = slab_ref[:, k * _TN:(k + 1) * _TN]
        sg = seg_ref[:, k * _TN:(k + 1) * _TN]
        base = base_ref[(i * half + j) * sub + k]
        win = lax.broadcasted_iota(jnp.int32, (w, _TN), 0) + base
        mask = (win == sg).astype(jnp.float32)
        psum = jnp.dot(mask, sl.T, preferred_element_type=jnp.float32)
        b8 = pl.multiple_of(base, 8)
        if True:
            pass
        if (k % 2) == 0:
            out_ref[0, pl.ds(b8, w), :] += psum
        else:
            acc1[pl.ds(b8, w), :] += psum

    @pl.when(j == half - 1)
    def _():
        out_ref[0] = out_ref[0] + acc1[...]


def _pfn_kernel(base_ref, slab_ref, seg_ref, first_ref, sums_ref,
                wpt_ref, wmn_ref, out_ref, mean_ref, acc1):
    i = pl.program_id(0)
    j = pl.program_id(1)
    half = pl.num_programs(1)
    tnb = slab_ref.shape[1]
    sub = tnb // _TN
    w = _TN + 8

    @pl.when(j == 0)
    def _():
        s = sums_ref[0] + sums_ref[1]
        cnt = jnp.maximum(s[:, _SLAB - 1:_SLAB], 1.0)
        mean_ref[...] = s / cnt
        out_ref[...] = jnp.zeros_like(out_ref)
        acc1[...] = jnp.zeros_like(acc1)

    for k in range(sub):
        sl = slab_ref[:, k * _TN:(k + 1) * _TN]
        sg = seg_ref[:, k * _TN:(k + 1) * _TN]
        fr = first_ref[:, k * _TN:(k + 1) * _TN]
        base = base_ref[(i * half + j) * sub + k]
        b8 = pl.multiple_of(base, 8)
        win = lax.broadcasted_iota(jnp.int32, (w, _TN), 0) + base
        mask = (win == sg).astype(jnp.float32)

        mean_t = mean_ref[pl.ds(b8, w), :].T
        pm = jnp.dot(mean_t, mask, preferred_element_type=jnp.float32)
        pf = jnp.maximum(
            jnp.dot(wpt_ref[...], sl, preferred_element_type=jnp.float32)
            + jnp.dot(wmn_ref[...], pm, preferred_element_type=jnp.float32),
            0.0)

        kk = 1
        while kk < _TN:
            sh = _TN - kk
            same = sg == pltpu.roll(sg, shift=sh, axis=1)
            pf = jnp.where(same,
                           jnp.maximum(pf, pltpu.roll(pf, shift=sh, axis=1)),
                           pf)
            kk *= 2

        lane = lax.broadcasted_iota(jnp.int32, (1, _TN), 1)
        sel = jnp.logical_or(fr != 0, lane == 0)
        pf_sel = jnp.where(sel, pf, 0.0)

        loc = jnp.dot(mask, pf_sel.T,
                      preferred_element_type=jnp.float32)
        if (k % 2) == 0:
            out_ref[0, pl.ds(b8, w), :] = jnp.maximum(
                out_ref[0, pl.ds(b8, w), :], loc)
        else:
            acc1[pl.ds(b8, w), :] = jnp.maximum(acc1[pl.ds(b8, w), :], loc)

    @pl.when(j == half - 1)
    def _():
        out_ref[0] = jnp.maximum(out_ref[0], acc1[...])


def _dpfn(features, coors, w_t, bn_scale, bn_shift, *,
          voxel_size, point_cloud_range, v_max, v_acc, tn, sub):
    n = features.shape[0]
    c_out = w_t.shape[1]
    tnb = tn * sub

    vx, vy, vz = (float(s) for s in voxel_size)
    x_off = vx / 2.0 + float(point_cloud_range[0])
    y_off = vy / 2.0 + float(point_cloud_range[1])
    z_off = vz / 2.0 + float(point_cloud_range[2])
    canvas_y = int(round((point_cloud_range[4] - point_cloud_range[1]) / vy))
    canvas_x = int(round((point_cloud_range[3] - point_cloud_range[0]) / vx))

    coors = coors.astype(jnp.int32)
    feats = features.astype(jnp.float32)

    vid = (coors[:, 0] * canvas_y * canvas_x
           + coors[:, 2] * canvas_x + coors[:, 3])
    order = jnp.argsort(vid)
    vid_s = vid[order]
    feats_s = feats[order]
    is_new = jnp.concatenate(
        [jnp.ones((1,), jnp.bool_), vid_s[1:] != vid_s[:-1]])
    seg = jnp.cumsum(is_new.astype(jnp.int32)) - 1

    cb = (vid_s // (canvas_y * canvas_x)).astype(jnp.float32)
    cy = ((vid_s // canvas_x) % canvas_y).astype(jnp.float32)
    cx = (vid_s % canvas_x).astype(jnp.float32)

    fcx = feats_s[:, 0] - (cx * vx + x_off)
    fcy = feats_s[:, 1] - (cy * vy + y_off)
    slab_t = jnp.stack([feats_s[:, 0], feats_s[:, 1], feats_s[:, 2],
                        feats_s[:, 3], fcx, fcy, cb,
                        jnp.ones((n,), jnp.float32)], axis=0)

    bn_scale = bn_scale.reshape(1, -1).astype(jnp.float32)
    bn_shift = bn_shift.reshape(-1).astype(jnp.float32)
    w_eff = w_t.astype(jnp.float32) * bn_scale
    w_pt = jnp.stack([w_eff[0] + w_eff[4], w_eff[1] + w_eff[5],
                      w_eff[2] + w_eff[6] + w_eff[9], w_eff[3],
                      w_eff[7], w_eff[8],
                      jnp.zeros((c_out,), jnp.float32),
                      bn_shift - z_off * w_eff[9]], axis=0).T
    w_mn = jnp.concatenate(
        [-w_eff[4:7], jnp.zeros((_SLAB - 3, c_out), jnp.float32)], axis=0).T

    n_pad = _ceil_to(max(n, tnb), 2 * tnb)
    sentinel = jnp.int32(1 << 24)
    slab_t = jnp.pad(slab_t, ((0, 0), (0, n_pad - n)))
    seg_p = jnp.pad(seg, (0, n_pad - n), constant_values=sentinel)
    seg_row = seg_p.reshape(1, n_pad)
    first_row = jnp.pad(is_new.astype(jnp.int32),
                        (0, n_pad - n)).reshape(1, n_pad)
    bases = jnp.minimum((seg_p[::tn] // 8) * 8, v_max).astype(jnp.int32)

    blocks = n_pad // tnb
    grid = (2, blocks // 2)
    cparams = pltpu.CompilerParams(
        dimension_semantics=("parallel", "arbitrary"),
        vmem_limit_bytes=100 * 1024 * 1024)

    base_spec = pl.BlockSpec(memory_space=pltpu.SMEM)
    slab_spec = pl.BlockSpec((_SLAB, tnb),
                             lambda i, j: (0, i * (blocks // 2) + j))
    seg_spec = pl.BlockSpec((1, tnb), lambda i, j: (0, i * (blocks // 2) + j))
    acc8_spec = pl.BlockSpec((1, v_acc, _SLAB), lambda i, j: (i, 0, 0))
    sums_spec = pl.BlockSpec((2, v_acc, _SLAB), lambda i, j: (0, 0, 0))
    w_spec = pl.BlockSpec((c_out, _SLAB), lambda i, j: (0, 0))
    out_spec = pl.BlockSpec((1, v_acc, c_out), lambda i, j: (i, 0, 0))

    sums = pl.pallas_call(
        _sum_kernel,
        out_shape=jax.ShapeDtypeStruct((2, v_acc, _SLAB), jnp.float32),
        grid=grid,
        in_specs=[base_spec, slab_spec, seg_spec],
        out_specs=acc8_spec,
        scratch_shapes=[pltpu.VMEM((v_acc, _SLAB), jnp.float32)],
        compiler_params=cparams,
    )(bases, slab_t, seg_row)

    feats_p = pl.pallas_call(
        _pfn_kernel,
        out_shape=jax.ShapeDtypeStruct((2, v_acc, c_out), jnp.float32),
        grid=grid,
        in_specs=[base_spec, slab_spec, seg_spec, seg_spec, sums_spec,
                  w_spec, w_spec],
        out_specs=out_spec,
        scratch_shapes=[pltpu.VMEM((v_acc, _SLAB), jnp.float32),
                        pltpu.VMEM((v_acc, c_out), jnp.float32)],
        compiler_params=cparams,
    )(bases, slab_t, seg_row, first_row, sums, w_pt, w_mn)

    voxel_feats = jnp.maximum(feats_p[0], feats_p[1])[:v_max]

    s = sums[0, :v_max] + sums[1, :v_max]
    cnt = s[:, _SLAB - 1]
    occ = cnt > 0.0
    safe = jnp.maximum(cnt, 1.0)
    vcx = jnp.round(((s[:, 0] - s[:, 4]) / safe - x_off) / vx).astype(jnp.int32)
    vcy = jnp.round(((s[:, 1] - s[:, 5]) / safe - y_off) / vy).astype(jnp.int32)
    vcb = jnp.round(s[:, 6] / safe).astype(jnp.int32)
    voxel_coors = jnp.where(
        occ[:, None],
        jnp.stack([vcb, jnp.zeros_like(vcb), vcy, vcx], axis=-1),
        jnp.int32(-1))
    return voxel_feats, voxel_coors


def kernel(features, coors, w_t, bn_scale, bn_shift):
    return _dpfn(features, coors, w_t, bn_scale, bn_shift,
                 voxel_size=(0.16, 0.16, 4.0),
                 point_cloud_range=(0.0, -6.4, -3.0, 20.48, 6.4, 1.0),
                 v_max=_VMAX, v_acc=_VACC, tn=_TN, sub=_SUB)
